# baseline (speedup 1.0000x reference)
import jax
import jax.numpy as jnp
from jax.experimental import pallas as pl

_EMBED_DIM = [64, 128, 256, 512]
_NUM_NODES = 4096
_HEADS = 2
_KNN_K = [20, 20, 20, 8]
_NSAMPLE = 16


def _sqdist(a, b):
    return (jnp.sum(a * a, -1)[:, :, None] + jnp.sum(b * b, -1)[:, None, :]
            - 2.0 * jnp.einsum('bnd,bmd->bnm', a, b))


def _knn_idx(query, ref, k):
    d2 = _sqdist(query, ref)
    _, idx = jax.lax.top_k(-d2, k)
    return idx


def _gath(x, idx):
    return jax.vmap(lambda xx, ii: xx[ii])(x, idx)


def _sa_block(x, coords, Wqkv, Wo, k, heads):
    b, N, d = x.shape
    qkv = x @ Wqkv
    q, kk, vv = jnp.split(qkv, 3, axis=-1)
    idx = _knn_idx(coords, coords, k)
    kn = _gath(kk, idx)
    vn = _gath(vv, idx)
    dh = d // heads
    q = q.reshape(b, N, heads, dh)
    kn = kn.reshape(b, N, k, heads, dh)
    vn = vn.reshape(b, N, k, heads, dh)
    attn = jnp.einsum('bnhd,bnkhd->bnhk', q, kn) / jnp.sqrt(jnp.float32(dh))
    attn = jax.nn.softmax(attn, axis=-1)
    out = jnp.einsum('bnhk,bnkhd->bnhd', attn, vn).reshape(b, N, d)
    return x + out @ Wo


def _fps_probe(coords, npoint):
    def one(c):
        N = c.shape[0]
        def body(i, state):
            sel, dist, last = state
            d2 = jnp.sum((c - c[last]) ** 2, axis=-1)
            dist = jnp.minimum(dist, d2)
            nxt = jnp.argmax(dist).astype(jnp.int32)
            sel = sel.at[i].set(nxt)
            return (sel, dist, nxt)
        sel0 = jnp.zeros((npoint,), jnp.int32)
        dist0 = jnp.full((N,), 1e10, dtype=jnp.float32)
        sel, _, _ = jax.lax.fori_loop(1, npoint, body, (sel0, dist0, jnp.int32(0)))
        return sel
    return jax.vmap(one)(jax.lax.stop_gradient(coords))


def _fps_embed(coords, x, npoint, W, b_):
    sel = _fps_probe(coords, npoint)
    new_coords = jax.vmap(lambda c, s: c[s])(coords, sel)
    idx = _knn_idx(new_coords, coords, _NSAMPLE)
    gx = _gath(x, idx)
    gc = _gath(coords, idx)
    rel = gc - new_coords[:, :, None, :]
    feat = jnp.concatenate([gx, rel], axis=-1) @ W + b_
    feat = jax.nn.relu(feat)
    feat = jnp.max(feat, axis=2)
    return feat, new_coords


def _copy_kernel(x_ref, o_ref):
    o_ref[...] = x_ref[...]


def kernel(coords, x, W_stfe, b_stfe, blk0_Wqkv, blk0_Wo, blk1_Wqkv, blk1_Wo,
           blk2_Wqkv, blk2_Wo, blk3_Wqkv, blk3_Wo, fps0_W, fps0_b, fps1_W,
           fps1_b, fps2_W, fps2_b):
    blk_Wqkv = [blk0_Wqkv, blk1_Wqkv, blk2_Wqkv, blk3_Wqkv]
    blk_Wo = [blk0_Wo, blk1_Wo, blk2_Wo, blk3_Wo]
    fps_W = [fps0_W, fps1_W, fps2_W]
    fps_b = [fps0_b, fps1_b, fps2_b]
    b, N = x.shape[0], x.shape[1]
    h = x.reshape(b, N, -1) @ W_stfe + b_stfe
    c = coords[:, :, 0:3]
    for i in range(len(_EMBED_DIM)):
        h = _sa_block(h, c, blk_Wqkv[i], blk_Wo[i], _KNN_K[i], _HEADS)
        if i != len(_EMBED_DIM) - 1:
            h, c = _fps_embed(c, h, _NUM_NODES // 4 ** (i + 1), fps_W[i], fps_b[i])
    out = pl.pallas_call(
        _copy_kernel,
        out_shape=jax.ShapeDtypeStruct(h.shape, h.dtype),
    )(h)
    return out


# speedup vs baseline: 9.0805x; 9.0805x over previous
import functools
import math

import jax
import jax.numpy as jnp
from jax.experimental import pallas as pl

_EMBED = [64, 128, 256, 512]
_N0 = 4096
_HEADS = 2
_KNN = [20, 20, 20, 8]
_NS = 16
_HI = jax.lax.Precision.HIGHEST
_NEG = -1e30


def _mm_kern(x_ref, w_ref, o_ref):
    o_ref[...] = jnp.dot(x_ref[...], w_ref[...],
                         preferred_element_type=jnp.float32, precision=_HI)


def _mm_bias_kern(x_ref, w_ref, b_ref, o_ref):
    o_ref[...] = (jnp.dot(x_ref[...], w_ref[...],
                          preferred_element_type=jnp.float32, precision=_HI)
                  + b_ref[...])


def _mm(x, w):
    return pl.pallas_call(
        _mm_kern,
        out_shape=jax.ShapeDtypeStruct((x.shape[0], w.shape[1]), jnp.float32),
    )(x, w)


def _mm_bias(x, w, b):
    return pl.pallas_call(
        _mm_bias_kern,
        out_shape=jax.ShapeDtypeStruct((x.shape[0], w.shape[1]), jnp.float32),
    )(x, w, b.reshape(1, -1))


def _pv_kern(h_ref, c_ref, wx_ref, wc_ref, o_ref):
    o_ref[...] = (jnp.dot(h_ref[...], wx_ref[...],
                          preferred_element_type=jnp.float32, precision=_HI)
                  + jnp.dot(c_ref[...], wc_ref[...],
                            preferred_element_type=jnp.float32, precision=_HI))


def _pv(h2, c2, wx, wc):
    return pl.pallas_call(
        _pv_kern,
        out_shape=jax.ShapeDtypeStruct((h2.shape[0], wx.shape[1]), jnp.float32),
    )(h2, c2, wx, wc)


def _attn_body(k, heads, d, R, N,
               q_ref, h_ref, kT_ref, v_ref, cq_ref, cT_ref, wo_ref, o_ref):
    dh = d // heads
    scale = 1.0 / math.sqrt(float(dh))
    cq = cq_ref[0]
    cT = cT_ref[0]
    n2q = jnp.sum(cq * cq, axis=1, keepdims=True)
    n2m = jnp.sum(cT * cT, axis=0, keepdims=True)
    d2 = (n2q + n2m
          - 2.0 * jnp.dot(cq.astype(jnp.bfloat16), cT.astype(jnp.bfloat16),
                          preferred_element_type=jnp.float32))
    g = jnp.full((R, 1), -jnp.inf, jnp.float32)
    for _ in range(k):
        g = jnp.min(jnp.where(d2 > g, d2, jnp.inf), axis=1, keepdims=True)
    mask = d2 <= g
    q = q_ref[0]
    v = v_ref[0]
    kT = kT_ref[0]
    outs = []
    for hh in range(heads):
        qh = q[:, hh * dh:(hh + 1) * dh]
        kTh = kT[hh * dh:(hh + 1) * dh, :]
        vh = v[:, hh * dh:(hh + 1) * dh]
        lg = jnp.dot(qh, kTh, preferred_element_type=jnp.float32,
                     precision=_HI) * scale
        ml = jnp.where(mask, lg, _NEG)
        mx = jnp.max(ml, axis=1, keepdims=True)
        e = jnp.where(mask, jnp.exp(ml - mx), 0.0)
        s = jnp.sum(e, axis=1, keepdims=True)
        outs.append(jnp.dot(e / s, vh, preferred_element_type=jnp.float32,
                            precision=_HI))
    out = jnp.concatenate(outs, axis=1)
    o_ref[0] = h_ref[0] + jnp.dot(out, wo_ref[...],
                                  preferred_element_type=jnp.float32,
                                  precision=_HI)


def _attention(h, c, wqkv, wo, k, R):
    b, N, d = h.shape
    qkv = _mm(h.reshape(b * N, d), wqkv).reshape(b, N, 3 * d)
    q = qkv[:, :, 0:d]
    kk = qkv[:, :, d:2 * d]
    v = qkv[:, :, 2 * d:3 * d]
    kT = jnp.swapaxes(kk, 1, 2)
    cT = jnp.swapaxes(c, 1, 2)
    body = functools.partial(_attn_body, k, _HEADS, d, R, N)
    return pl.pallas_call(
        body,
        grid=(b, N // R),
        in_specs=[
            pl.BlockSpec((1, R, d), lambda bi, ri: (bi, ri, 0)),
            pl.BlockSpec((1, R, d), lambda bi, ri: (bi, ri, 0)),
            pl.BlockSpec((1, d, N), lambda bi, ri: (bi, 0, 0)),
            pl.BlockSpec((1, N, d), lambda bi, ri: (bi, 0, 0)),
            pl.BlockSpec((1, R, 3), lambda bi, ri: (bi, ri, 0)),
            pl.BlockSpec((1, 3, N), lambda bi, ri: (bi, 0, 0)),
            pl.BlockSpec((d, d), lambda bi, ri: (0, 0)),
        ],
        out_specs=pl.BlockSpec((1, R, d), lambda bi, ri: (bi, ri, 0)),
        out_shape=jax.ShapeDtypeStruct((b, N, d), jnp.float32),
    )(q, h, kT, v, c, cT, wo)


def _fps_body(npoint, N, cp_ref, c_ref, nc_ref):
    sub = N // 128
    lane = jax.lax.broadcasted_iota(jnp.int32, (sub, 128), 1)
    slid = jax.lax.broadcasted_iota(jnp.int32, (sub, 128), 0)
    flat = slid * 128 + lane
    big = jnp.int32(2 ** 31 - 1)
    for bi in range(2):
        nc_ref[bi, 0:1, :] = c_ref[bi, 0:1, :]

    def body(i, state):
        new_state = []
        for bi in range(2):
            dist, last = state[bi]
            row = c_ref[bi, pl.ds(last, 1), :]
            dx = cp_ref[bi, 0] - row[0:1, 0:1]
            dy = cp_ref[bi, 1] - row[0:1, 1:2]
            dz = cp_ref[bi, 2] - row[0:1, 2:3]
            d2 = dx * dx + dy * dy + dz * dz
            dist = jnp.minimum(dist, d2)
            m = jnp.max(dist)
            nxt = jnp.min(jnp.where(dist == m, flat, big)).astype(jnp.int32)
            nc_ref[bi, pl.ds(i, 1), :] = c_ref[bi, pl.ds(nxt, 1), :]
            new_state.append((dist, nxt))
        return tuple(new_state)

    dist0 = jnp.full((sub, 128), 1e10, jnp.float32)
    init = ((dist0, jnp.int32(0)), (dist0, jnp.int32(0)))
    jax.lax.fori_loop(1, npoint, body, init)


def _fps(c, npoint):
    b, N, _ = c.shape
    cp = jnp.swapaxes(c, 1, 2).reshape(b, 3, N // 128, 128)
    body = functools.partial(_fps_body, npoint, N)
    return pl.pallas_call(
        body,
        out_shape=jax.ShapeDtypeStruct((b, npoint, 3), jnp.float32),
    )(cp, c)


def _feat_body(ns, dout, Rq, N,
               nc_ref, cT_ref, pv_ref, wc_ref, b_ref, o_ref):
    nc = nc_ref[0]
    cT = cT_ref[0]
    n2q = jnp.sum(nc * nc, axis=1, keepdims=True)
    n2m = jnp.sum(cT * cT, axis=0, keepdims=True)
    d2 = (n2q + n2m
          - 2.0 * jnp.dot(nc.astype(jnp.bfloat16), cT.astype(jnp.bfloat16),
                          preferred_element_type=jnp.float32))
    pv = pv_ref[0]
    g = jnp.full((Rq, 1), -jnp.inf, jnp.float32)
    acc = jnp.full((Rq, dout), _NEG, jnp.float32)
    for _ in range(ns):
        g = jnp.min(jnp.where(d2 > g, d2, jnp.inf), axis=1, keepdims=True)
        e = (d2 == g).astype(jnp.float32)
        acc = jnp.maximum(acc, jnp.dot(e, pv,
                                       preferred_element_type=jnp.float32,
                                       precision=_HI))
    ncp = jnp.dot(nc, wc_ref[...], preferred_element_type=jnp.float32,
                  precision=_HI)
    o_ref[0] = jnp.maximum(acc - ncp + b_ref[...], 0.0)


def _fps_embed(c, h, npoint, w, bb, Rq):
    b, N, d = h.shape
    dout = w.shape[1]
    wx = w[0:d]
    wc = w[d:d + 3]
    nc = _fps(c, npoint)
    pv = _pv(h.reshape(b * N, d), c.reshape(b * N, 3),
             wx, wc).reshape(b, N, dout)
    cT = jnp.swapaxes(c, 1, 2)
    body = functools.partial(_feat_body, _NS, dout, Rq, N)
    feat = pl.pallas_call(
        body,
        grid=(b, npoint // Rq),
        in_specs=[
            pl.BlockSpec((1, Rq, 3), lambda bi, ri: (bi, ri, 0)),
            pl.BlockSpec((1, 3, N), lambda bi, ri: (bi, 0, 0)),
            pl.BlockSpec((1, N, dout), lambda bi, ri: (bi, 0, 0)),
            pl.BlockSpec((3, dout), lambda bi, ri: (0, 0)),
            pl.BlockSpec((1, dout), lambda bi, ri: (0, 0)),
        ],
        out_specs=pl.BlockSpec((1, Rq, dout), lambda bi, ri: (bi, ri, 0)),
        out_shape=jax.ShapeDtypeStruct((b, npoint, dout), jnp.float32),
    )(nc, cT, pv, wc, bb.reshape(1, dout))
    return feat, nc


_ATTN_R = [512, 512, 256, 64]
_FEAT_R = [256, 256, 64]


def kernel(coords, x, W_stfe, b_stfe, blk0_Wqkv, blk0_Wo, blk1_Wqkv, blk1_Wo,
           blk2_Wqkv, blk2_Wo, blk3_Wqkv, blk3_Wo, fps0_W, fps0_b, fps1_W,
           fps1_b, fps2_W, fps2_b):
    wqkv = [blk0_Wqkv, blk1_Wqkv, blk2_Wqkv, blk3_Wqkv]
    wo = [blk0_Wo, blk1_Wo, blk2_Wo, blk3_Wo]
    fw = [fps0_W, fps1_W, fps2_W]
    fb = [fps0_b, fps1_b, fps2_b]
    b, N = x.shape[0], x.shape[1]
    h = _mm_bias(x.reshape(b * N, -1), W_stfe, b_stfe).reshape(b, N, _EMBED[0])
    c = coords[:, :, 0:3]
    for i in range(4):
        h = _attention(h, c, wqkv[i], wo[i], _KNN[i], _ATTN_R[i])
        if i != 3:
            npoint = _N0 // 4 ** (i + 1)
            h, c = _fps_embed(c, h, npoint, fw[i], fb[i], _FEAT_R[i])
            N = npoint
    return h


# speedup vs baseline: 11.3239x; 1.2471x over previous
import functools
import math

import jax
import jax.numpy as jnp
from jax.experimental import pallas as pl
from jax.experimental.pallas import tpu as pltpu

_EMBED = [64, 128, 256, 512]
_N0 = 4096
_HEADS = 2
_KNN = [20, 20, 20, 8]
_NS = 16
_HI = jax.lax.Precision.HIGHEST
_NEG = -1e30
_BF = jnp.bfloat16


def _bdot(a, b):
    return jnp.dot(a.astype(_BF), b.astype(_BF),
                   preferred_element_type=jnp.float32)


def _mm_kern(x_ref, w_ref, o_ref):
    o_ref[...] = _bdot(x_ref[...], w_ref[...])


def _mm_bias_kern(x_ref, w_ref, b_ref, o_ref):
    o_ref[...] = _bdot(x_ref[...], w_ref[...]) + b_ref[...]


def _par(n):
    return pltpu.CompilerParams(dimension_semantics=("parallel",) * n)


def _mm(x, w):
    m = x.shape[0]
    return pl.pallas_call(
        _mm_kern,
        grid=(2,),
        in_specs=[
            pl.BlockSpec((m // 2, x.shape[1]), lambda i: (i, 0)),
            pl.BlockSpec((x.shape[1], w.shape[1]), lambda i: (0, 0)),
        ],
        out_specs=pl.BlockSpec((m // 2, w.shape[1]), lambda i: (i, 0)),
        out_shape=jax.ShapeDtypeStruct((m, w.shape[1]), jnp.float32),
        compiler_params=_par(1),
    )(x, w)


def _mm_bias(x, w, b):
    m = x.shape[0]
    return pl.pallas_call(
        _mm_bias_kern,
        grid=(2,),
        in_specs=[
            pl.BlockSpec((m // 2, x.shape[1]), lambda i: (i, 0)),
            pl.BlockSpec((x.shape[1], w.shape[1]), lambda i: (0, 0)),
            pl.BlockSpec((1, w.shape[1]), lambda i: (0, 0)),
        ],
        out_specs=pl.BlockSpec((m // 2, w.shape[1]), lambda i: (i, 0)),
        out_shape=jax.ShapeDtypeStruct((m, w.shape[1]), jnp.float32),
        compiler_params=_par(1),
    )(x, w, b.reshape(1, -1))


def _pv_kern(h_ref, c_ref, wx_ref, wc_ref, o_ref):
    o_ref[...] = _bdot(h_ref[...], wx_ref[...]) + _bdot(c_ref[...], wc_ref[...])


def _pv(h2, c2, wx, wc):
    m = h2.shape[0]
    return pl.pallas_call(
        _pv_kern,
        grid=(2,),
        in_specs=[
            pl.BlockSpec((m // 2, h2.shape[1]), lambda i: (i, 0)),
            pl.BlockSpec((m // 2, 3), lambda i: (i, 0)),
            pl.BlockSpec((h2.shape[1], wx.shape[1]), lambda i: (0, 0)),
            pl.BlockSpec((3, wx.shape[1]), lambda i: (0, 0)),
        ],
        out_specs=pl.BlockSpec((m // 2, wx.shape[1]), lambda i: (i, 0)),
        out_shape=jax.ShapeDtypeStruct((m, wx.shape[1]), jnp.float32),
        compiler_params=_par(1),
    )(h2, c2, wx, wc)


def _attn_body(k, heads, d, R, N,
               q_ref, h_ref, kT_ref, v_ref, cq_ref, cT_ref, wo_ref, o_ref):
    dh = d // heads
    scale = 1.0 / math.sqrt(float(dh))
    cq = cq_ref[0]
    cT = cT_ref[0]
    n2q = jnp.sum(cq * cq, axis=1, keepdims=True)
    n2m = jnp.sum(cT * cT, axis=0, keepdims=True)
    d2 = n2q + n2m - 2.0 * _bdot(cq, cT)
    g = jnp.full((R, 1), -jnp.inf, jnp.float32)
    for _ in range(k):
        g = jnp.min(jnp.where(d2 > g, d2, jnp.inf), axis=1, keepdims=True)
    mask = d2 <= g
    q = q_ref[0]
    v = v_ref[0]
    kT = kT_ref[0]
    outs = []
    for hh in range(heads):
        qh = q[:, hh * dh:(hh + 1) * dh]
        kTh = kT[hh * dh:(hh + 1) * dh, :]
        vh = v[:, hh * dh:(hh + 1) * dh]
        lg = _bdot(qh, kTh) * scale
        ml = jnp.where(mask, lg, _NEG)
        mx = jnp.max(ml, axis=1, keepdims=True)
        e = jnp.where(mask, jnp.exp(ml - mx), 0.0)
        s = jnp.sum(e, axis=1, keepdims=True)
        outs.append(_bdot(e / s, vh))
    out = jnp.concatenate(outs, axis=1)
    o_ref[0] = h_ref[0] + _bdot(out, wo_ref[...])


def _attention(h, c, wqkv, wo, k, R):
    b, N, d = h.shape
    qkv = _mm(h.reshape(b * N, d), wqkv).reshape(b, N, 3 * d)
    q = qkv[:, :, 0:d]
    kk = qkv[:, :, d:2 * d]
    v = qkv[:, :, 2 * d:3 * d]
    kT = jnp.swapaxes(kk, 1, 2)
    cT = jnp.swapaxes(c, 1, 2)
    body = functools.partial(_attn_body, k, _HEADS, d, R, N)
    return pl.pallas_call(
        body,
        grid=(b, N // R),
        in_specs=[
            pl.BlockSpec((1, R, d), lambda bi, ri: (bi, ri, 0)),
            pl.BlockSpec((1, R, d), lambda bi, ri: (bi, ri, 0)),
            pl.BlockSpec((1, d, N), lambda bi, ri: (bi, 0, 0)),
            pl.BlockSpec((1, N, d), lambda bi, ri: (bi, 0, 0)),
            pl.BlockSpec((1, R, 3), lambda bi, ri: (bi, ri, 0)),
            pl.BlockSpec((1, 3, N), lambda bi, ri: (bi, 0, 0)),
            pl.BlockSpec((d, d), lambda bi, ri: (0, 0)),
        ],
        out_specs=pl.BlockSpec((1, R, d), lambda bi, ri: (bi, ri, 0)),
        out_shape=jax.ShapeDtypeStruct((b, N, d), jnp.float32),
        compiler_params=_par(2),
    )(q, h, kT, v, c, cT, wo)


def _fps_body(npoint, N, cp_ref, c_ref, nc_ref):
    sub = N // 128
    lane = jax.lax.broadcasted_iota(jnp.int32, (sub, 128), 1)
    slid = jax.lax.broadcasted_iota(jnp.int32, (sub, 128), 0)
    flat = slid * 128 + lane
    big = jnp.int32(2 ** 31 - 1)
    nc_ref[0, 0:1, :] = c_ref[0, 0:1, :]

    def body(i, state):
        dist, last = state
        row = c_ref[0, pl.ds(last, 1), :]
        dx = cp_ref[0, 0] - row[0:1, 0:1]
        dy = cp_ref[0, 1] - row[0:1, 1:2]
        dz = cp_ref[0, 2] - row[0:1, 2:3]
        d2 = dx * dx + dy * dy + dz * dz
        dist = jnp.minimum(dist, d2)
        m = jnp.max(dist)
        nxt = jnp.min(jnp.where(dist == m, flat, big)).astype(jnp.int32)
        nc_ref[0, pl.ds(i, 1), :] = c_ref[0, pl.ds(nxt, 1), :]
        return (dist, nxt)

    dist0 = jnp.full((sub, 128), 1e10, jnp.float32)
    jax.lax.fori_loop(1, npoint, body, (dist0, jnp.int32(0)))


def _fps(c, npoint):
    b, N, _ = c.shape
    cp = jnp.swapaxes(c, 1, 2).reshape(b, 3, N // 128, 128)
    body = functools.partial(_fps_body, npoint, N)
    return pl.pallas_call(
        body,
        grid=(b,),
        in_specs=[
            pl.BlockSpec((1, 3, N // 128, 128), lambda bi: (bi, 0, 0, 0)),
            pl.BlockSpec((1, N, 3), lambda bi: (bi, 0, 0)),
        ],
        out_specs=pl.BlockSpec((1, npoint, 3), lambda bi: (bi, 0, 0)),
        out_shape=jax.ShapeDtypeStruct((b, npoint, 3), jnp.float32),
        compiler_params=_par(1),
    )(cp, c)


def _feat_body(ns, dout, Rq, N,
               nc_ref, cT_ref, pv_ref, wc_ref, b_ref, o_ref):
    nc = nc_ref[0]
    cT = cT_ref[0]
    n2q = jnp.sum(nc * nc, axis=1, keepdims=True)
    n2m = jnp.sum(cT * cT, axis=0, keepdims=True)
    d2 = n2q + n2m - 2.0 * _bdot(nc, cT)
    pv = pv_ref[0]
    g = jnp.full((Rq, 1), -jnp.inf, jnp.float32)
    acc = jnp.full((Rq, dout), _NEG, jnp.float32)
    for _ in range(ns):
        g = jnp.min(jnp.where(d2 > g, d2, jnp.inf), axis=1, keepdims=True)
        e = (d2 == g).astype(jnp.float32)
        acc = jnp.maximum(acc, jnp.dot(e, pv,
                                       preferred_element_type=jnp.float32,
                                       precision=_HI))
    ncp = _bdot(nc, wc_ref[...])
    o_ref[0] = jnp.maximum(acc - ncp + b_ref[...], 0.0)


def _fps_embed(c, h, npoint, w, bb, Rq):
    b, N, d = h.shape
    dout = w.shape[1]
    wx = w[0:d]
    wc = w[d:d + 3]
    nc = _fps(c, npoint)
    pv = _pv(h.reshape(b * N, d), c.reshape(b * N, 3),
             wx, wc).reshape(b, N, dout)
    cT = jnp.swapaxes(c, 1, 2)
    body = functools.partial(_feat_body, _NS, dout, Rq, N)
    feat = pl.pallas_call(
        body,
        grid=(b, npoint // Rq),
        in_specs=[
            pl.BlockSpec((1, Rq, 3), lambda bi, ri: (bi, ri, 0)),
            pl.BlockSpec((1, 3, N), lambda bi, ri: (bi, 0, 0)),
            pl.BlockSpec((1, N, dout), lambda bi, ri: (bi, 0, 0)),
            pl.BlockSpec((3, dout), lambda bi, ri: (0, 0)),
            pl.BlockSpec((1, dout), lambda bi, ri: (0, 0)),
        ],
        out_specs=pl.BlockSpec((1, Rq, dout), lambda bi, ri: (bi, ri, 0)),
        out_shape=jax.ShapeDtypeStruct((b, npoint, dout), jnp.float32),
        compiler_params=_par(2),
    )(nc, cT, pv, wc, bb.reshape(1, dout))
    return feat, nc


_ATTN_R = [512, 512, 256, 64]
_FEAT_R = [256, 256, 64]


def kernel(coords, x, W_stfe, b_stfe, blk0_Wqkv, blk0_Wo, blk1_Wqkv, blk1_Wo,
           blk2_Wqkv, blk2_Wo, blk3_Wqkv, blk3_Wo, fps0_W, fps0_b, fps1_W,
           fps1_b, fps2_W, fps2_b):
    wqkv = [blk0_Wqkv, blk1_Wqkv, blk2_Wqkv, blk3_Wqkv]
    wo = [blk0_Wo, blk1_Wo, blk2_Wo, blk3_Wo]
    fw = [fps0_W, fps1_W, fps2_W]
    fb = [fps0_b, fps1_b, fps2_b]
    b, N = x.shape[0], x.shape[1]
    h = _mm_bias(x.reshape(b * N, -1), W_stfe, b_stfe).reshape(b, N, _EMBED[0])
    c = coords[:, :, 0:3]
    for i in range(4):
        h = _attention(h, c, wqkv[i], wo[i], _KNN[i], _ATTN_R[i])
        if i != 3:
            npoint = _N0 // 4 ** (i + 1)
            h, c = _fps_embed(c, h, npoint, fw[i], fb[i], _FEAT_R[i])
            N = npoint
    return h


# speedup vs baseline: 11.9671x; 1.0568x over previous
import functools
import math

import jax
import jax.numpy as jnp
from jax.experimental import pallas as pl
from jax.experimental.pallas import tpu as pltpu

_EMBED = [64, 128, 256, 512]
_N0 = 4096
_HEADS = 2
_KNN = [20, 20, 20, 8]
_NS = 16
_HI = jax.lax.Precision.HIGHEST
_NEG = -1e30
_BF = jnp.bfloat16


def _bdot(a, b):
    return jnp.dot(a.astype(_BF), b.astype(_BF),
                   preferred_element_type=jnp.float32)


def _mm_kern(x_ref, w_ref, o_ref):
    o_ref[...] = _bdot(x_ref[...], w_ref[...])


def _mm_bias_kern(x_ref, w_ref, b_ref, o_ref):
    o_ref[...] = _bdot(x_ref[...], w_ref[...]) + b_ref[...]


def _par(n):
    return pltpu.CompilerParams(dimension_semantics=("parallel",) * n)


def _mm(x, w):
    m = x.shape[0]
    return pl.pallas_call(
        _mm_kern,
        grid=(2,),
        in_specs=[
            pl.BlockSpec((m // 2, x.shape[1]), lambda i: (i, 0)),
            pl.BlockSpec((x.shape[1], w.shape[1]), lambda i: (0, 0)),
        ],
        out_specs=pl.BlockSpec((m // 2, w.shape[1]), lambda i: (i, 0)),
        out_shape=jax.ShapeDtypeStruct((m, w.shape[1]), jnp.float32),
        compiler_params=_par(1),
    )(x, w)


def _mm_bias(x, w, b):
    m = x.shape[0]
    return pl.pallas_call(
        _mm_bias_kern,
        grid=(2,),
        in_specs=[
            pl.BlockSpec((m // 2, x.shape[1]), lambda i: (i, 0)),
            pl.BlockSpec((x.shape[1], w.shape[1]), lambda i: (0, 0)),
            pl.BlockSpec((1, w.shape[1]), lambda i: (0, 0)),
        ],
        out_specs=pl.BlockSpec((m // 2, w.shape[1]), lambda i: (i, 0)),
        out_shape=jax.ShapeDtypeStruct((m, w.shape[1]), jnp.float32),
        compiler_params=_par(1),
    )(x, w, b.reshape(1, -1))


def _pv_kern(h_ref, c_ref, wx_ref, wc_ref, o_ref):
    o_ref[...] = _bdot(h_ref[...], wx_ref[...]) + _bdot(c_ref[...], wc_ref[...])


def _pv(h2, c2, wx, wc):
    m = h2.shape[0]
    return pl.pallas_call(
        _pv_kern,
        grid=(2,),
        in_specs=[
            pl.BlockSpec((m // 2, h2.shape[1]), lambda i: (i, 0)),
            pl.BlockSpec((m // 2, 3), lambda i: (i, 0)),
            pl.BlockSpec((h2.shape[1], wx.shape[1]), lambda i: (0, 0)),
            pl.BlockSpec((3, wx.shape[1]), lambda i: (0, 0)),
        ],
        out_specs=pl.BlockSpec((m // 2, wx.shape[1]), lambda i: (i, 0)),
        out_shape=jax.ShapeDtypeStruct((m, wx.shape[1]), jnp.float32),
        compiler_params=_par(1),
    )(h2, c2, wx, wc)


def _attn_body(k, heads, d, R, N,
               q_ref, h_ref, kT_ref, v_ref, cq_ref, cT_ref, wo_ref, o_ref):
    dh = d // heads
    scale = 1.0 / math.sqrt(float(dh))
    cq = cq_ref[0]
    cT = cT_ref[0]
    n2q = jnp.sum(cq * cq, axis=1, keepdims=True)
    n2m = jnp.sum(cT * cT, axis=0, keepdims=True)
    d2 = n2q + n2m - 2.0 * _bdot(cq, cT)
    g = jnp.full((R, 1), -jnp.inf, jnp.float32)
    for _ in range(k):
        g = jnp.min(jnp.where(d2 > g, d2, jnp.inf), axis=1, keepdims=True)
    mask = d2 <= g
    q = q_ref[0]
    v = v_ref[0]
    kT = kT_ref[0]
    outs = []
    for hh in range(heads):
        qh = q[:, hh * dh:(hh + 1) * dh]
        kTh = kT[hh * dh:(hh + 1) * dh, :]
        vh = v[:, hh * dh:(hh + 1) * dh]
        lg = _bdot(qh, kTh) * scale
        ml = jnp.where(mask, lg, _NEG)
        mx = jnp.max(ml, axis=1, keepdims=True)
        e = jnp.where(mask, jnp.exp(ml - mx), 0.0)
        s = jnp.sum(e, axis=1, keepdims=True)
        outs.append(_bdot(e / s, vh))
    out = jnp.concatenate(outs, axis=1)
    o_ref[0] = h_ref[0] + _bdot(out, wo_ref[...])


def _attention(h, c, wqkv, wo, k, R):
    b, N, d = h.shape
    qkv = _mm(h.reshape(b * N, d), wqkv).reshape(b, N, 3 * d)
    q = qkv[:, :, 0:d]
    kk = qkv[:, :, d:2 * d]
    v = qkv[:, :, 2 * d:3 * d]
    kT = jnp.swapaxes(kk, 1, 2)
    cT = jnp.swapaxes(c, 1, 2)
    body = functools.partial(_attn_body, k, _HEADS, d, R, N)
    return pl.pallas_call(
        body,
        grid=(b, N // R),
        in_specs=[
            pl.BlockSpec((1, R, d), lambda bi, ri: (bi, ri, 0)),
            pl.BlockSpec((1, R, d), lambda bi, ri: (bi, ri, 0)),
            pl.BlockSpec((1, d, N), lambda bi, ri: (bi, 0, 0)),
            pl.BlockSpec((1, N, d), lambda bi, ri: (bi, 0, 0)),
            pl.BlockSpec((1, R, 3), lambda bi, ri: (bi, ri, 0)),
            pl.BlockSpec((1, 3, N), lambda bi, ri: (bi, 0, 0)),
            pl.BlockSpec((d, d), lambda bi, ri: (0, 0)),
        ],
        out_specs=pl.BlockSpec((1, R, d), lambda bi, ri: (bi, ri, 0)),
        out_shape=jax.ShapeDtypeStruct((b, N, d), jnp.float32),
        compiler_params=_par(2),
    )(q, h, kT, v, c, cT, wo)


def _fps_body(npoint, N, cp_ref, c_ref, nc_ref):
    sub = N // 128
    lane = jax.lax.broadcasted_iota(jnp.int32, (sub, 128), 1)
    slid = jax.lax.broadcasted_iota(jnp.int32, (sub, 128), 0)
    flat = slid * 128 + lane
    big = jnp.int32(2 ** 31 - 1)
    nc_ref[0, 0:1, :] = c_ref[0, 0:1, :]
    cx = cp_ref[0, 0]
    cy = cp_ref[0, 1]
    cz = cp_ref[0, 2]

    def body(i, state):
        dist, lx, ly, lz = state
        dx = cx - lx
        dy = cy - ly
        dz = cz - lz
        d2 = dx * dx + dy * dy + dz * dz
        dist = jnp.minimum(dist, d2)
        m = jnp.max(jnp.max(dist, axis=0, keepdims=True),
                    axis=1, keepdims=True)
        cand = jnp.where(dist == m, flat, big)
        nxt = jnp.min(jnp.min(cand, axis=0, keepdims=True),
                      axis=1, keepdims=True)
        hot = flat == nxt
        lx = jnp.sum(jnp.sum(jnp.where(hot, cx, 0.0), axis=0, keepdims=True),
                     axis=1, keepdims=True)
        ly = jnp.sum(jnp.sum(jnp.where(hot, cy, 0.0), axis=0, keepdims=True),
                     axis=1, keepdims=True)
        lz = jnp.sum(jnp.sum(jnp.where(hot, cz, 0.0), axis=0, keepdims=True),
                     axis=1, keepdims=True)
        nc_ref[0, pl.ds(i, 1), :] = jnp.concatenate([lx, ly, lz], axis=1)
        return (dist, lx, ly, lz)

    dist0 = jnp.full((sub, 128), 1e10, jnp.float32)
    jax.lax.fori_loop(1, npoint, body,
                      (dist0, cx[0:1, 0:1], cy[0:1, 0:1], cz[0:1, 0:1]))


def _fps(c, npoint):
    b, N, _ = c.shape
    cp = jnp.swapaxes(c, 1, 2).reshape(b, 3, N // 128, 128)
    body = functools.partial(_fps_body, npoint, N)
    return pl.pallas_call(
        body,
        grid=(b,),
        in_specs=[
            pl.BlockSpec((1, 3, N // 128, 128), lambda bi: (bi, 0, 0, 0)),
            pl.BlockSpec((1, N, 3), lambda bi: (bi, 0, 0)),
        ],
        out_specs=pl.BlockSpec((1, npoint, 3), lambda bi: (bi, 0, 0)),
        out_shape=jax.ShapeDtypeStruct((b, npoint, 3), jnp.float32),
        compiler_params=_par(1),
    )(cp, c)


def _feat_body(ns, dout, Rq, N,
               nc_ref, cT_ref, pv_ref, wc_ref, b_ref, o_ref):
    nc = nc_ref[0]
    cT = cT_ref[0]
    n2q = jnp.sum(nc * nc, axis=1, keepdims=True)
    n2m = jnp.sum(cT * cT, axis=0, keepdims=True)
    d2 = n2q + n2m - 2.0 * _bdot(nc, cT)
    pv = pv_ref[0]
    g = jnp.full((Rq, 1), -jnp.inf, jnp.float32)
    acc = jnp.full((Rq, dout), _NEG, jnp.float32)
    for _ in range(ns):
        g = jnp.min(jnp.where(d2 > g, d2, jnp.inf), axis=1, keepdims=True)
        e = (d2 == g).astype(jnp.float32)
        acc = jnp.maximum(acc, jnp.dot(e, pv,
                                       preferred_element_type=jnp.float32,
                                       precision=_HI))
    ncp = _bdot(nc, wc_ref[...])
    o_ref[0] = jnp.maximum(acc - ncp + b_ref[...], 0.0)


def _fps_embed(c, h, npoint, w, bb, Rq):
    b, N, d = h.shape
    dout = w.shape[1]
    wx = w[0:d]
    wc = w[d:d + 3]
    nc = _fps(c, npoint)
    pv = _pv(h.reshape(b * N, d), c.reshape(b * N, 3),
             wx, wc).reshape(b, N, dout)
    cT = jnp.swapaxes(c, 1, 2)
    body = functools.partial(_feat_body, _NS, dout, Rq, N)
    feat = pl.pallas_call(
        body,
        grid=(b, npoint // Rq),
        in_specs=[
            pl.BlockSpec((1, Rq, 3), lambda bi, ri: (bi, ri, 0)),
            pl.BlockSpec((1, 3, N), lambda bi, ri: (bi, 0, 0)),
            pl.BlockSpec((1, N, dout), lambda bi, ri: (bi, 0, 0)),
            pl.BlockSpec((3, dout), lambda bi, ri: (0, 0)),
            pl.BlockSpec((1, dout), lambda bi, ri: (0, 0)),
        ],
        out_specs=pl.BlockSpec((1, Rq, dout), lambda bi, ri: (bi, ri, 0)),
        out_shape=jax.ShapeDtypeStruct((b, npoint, dout), jnp.float32),
        compiler_params=_par(2),
    )(nc, cT, pv, wc, bb.reshape(1, dout))
    return feat, nc


_ATTN_R = [512, 512, 256, 64]
_FEAT_R = [256, 256, 64]


def kernel(coords, x, W_stfe, b_stfe, blk0_Wqkv, blk0_Wo, blk1_Wqkv, blk1_Wo,
           blk2_Wqkv, blk2_Wo, blk3_Wqkv, blk3_Wo, fps0_W, fps0_b, fps1_W,
           fps1_b, fps2_W, fps2_b):
    wqkv = [blk0_Wqkv, blk1_Wqkv, blk2_Wqkv, blk3_Wqkv]
    wo = [blk0_Wo, blk1_Wo, blk2_Wo, blk3_Wo]
    fw = [fps0_W, fps1_W, fps2_W]
    fb = [fps0_b, fps1_b, fps2_b]
    b, N = x.shape[0], x.shape[1]
    h = _mm_bias(x.reshape(b * N, -1), W_stfe, b_stfe).reshape(b, N, _EMBED[0])
    c = coords[:, :, 0:3]
    for i in range(4):
        h = _attention(h, c, wqkv[i], wo[i], _KNN[i], _ATTN_R[i])
        if i != 3:
            npoint = _N0 // 4 ** (i + 1)
            h, c = _fps_embed(c, h, npoint, fw[i], fb[i], _FEAT_R[i])
            N = npoint
    return h


# speedup vs baseline: 12.0863x; 1.0100x over previous
import functools
import math

import jax
import jax.numpy as jnp
from jax.experimental import pallas as pl
from jax.experimental.pallas import tpu as pltpu

_EMBED = [64, 128, 256, 512]
_N0 = 4096
_HEADS = 2
_KNN = [20, 20, 20, 8]
_NS = 16
_HI = jax.lax.Precision.HIGHEST
_NEG = -1e30
_BF = jnp.bfloat16


def _bdot(a, b):
    return jnp.dot(a.astype(_BF), b.astype(_BF),
                   preferred_element_type=jnp.float32)


def _mm_kern(x_ref, w_ref, o_ref):
    o_ref[...] = _bdot(x_ref[...], w_ref[...])


def _mm_bias_kern(x_ref, w_ref, b_ref, o_ref):
    o_ref[...] = _bdot(x_ref[...], w_ref[...]) + b_ref[...]


def _par(n):
    return pltpu.CompilerParams(dimension_semantics=("parallel",) * n)


def _mm(x, w):
    m = x.shape[0]
    return pl.pallas_call(
        _mm_kern,
        grid=(2,),
        in_specs=[
            pl.BlockSpec((m // 2, x.shape[1]), lambda i: (i, 0)),
            pl.BlockSpec((x.shape[1], w.shape[1]), lambda i: (0, 0)),
        ],
        out_specs=pl.BlockSpec((m // 2, w.shape[1]), lambda i: (i, 0)),
        out_shape=jax.ShapeDtypeStruct((m, w.shape[1]), jnp.float32),
        compiler_params=_par(1),
    )(x, w)


def _mm_bias(x, w, b):
    m = x.shape[0]
    return pl.pallas_call(
        _mm_bias_kern,
        grid=(2,),
        in_specs=[
            pl.BlockSpec((m // 2, x.shape[1]), lambda i: (i, 0)),
            pl.BlockSpec((x.shape[1], w.shape[1]), lambda i: (0, 0)),
            pl.BlockSpec((1, w.shape[1]), lambda i: (0, 0)),
        ],
        out_specs=pl.BlockSpec((m // 2, w.shape[1]), lambda i: (i, 0)),
        out_shape=jax.ShapeDtypeStruct((m, w.shape[1]), jnp.float32),
        compiler_params=_par(1),
    )(x, w, b.reshape(1, -1))


def _pv_kern(h_ref, c_ref, wx_ref, wc_ref, o_ref):
    o_ref[...] = _bdot(h_ref[...], wx_ref[...]) + _bdot(c_ref[...], wc_ref[...])


def _pv(h2, c2, wx, wc):
    m = h2.shape[0]
    return pl.pallas_call(
        _pv_kern,
        grid=(2,),
        in_specs=[
            pl.BlockSpec((m // 2, h2.shape[1]), lambda i: (i, 0)),
            pl.BlockSpec((m // 2, 3), lambda i: (i, 0)),
            pl.BlockSpec((h2.shape[1], wx.shape[1]), lambda i: (0, 0)),
            pl.BlockSpec((3, wx.shape[1]), lambda i: (0, 0)),
        ],
        out_specs=pl.BlockSpec((m // 2, wx.shape[1]), lambda i: (i, 0)),
        out_shape=jax.ShapeDtypeStruct((m, wx.shape[1]), jnp.float32),
        compiler_params=_par(1),
    )(h2, c2, wx, wc)


def _attn_body(k, heads, d, R, N,
               ht_ref, hf_ref, hT_ref, cq_ref, cT_ref, wqkv_ref, wo_ref,
               o_ref):
    dh = d // heads
    scale = 1.0 / math.sqrt(float(dh))
    cq = cq_ref[0]
    cT = cT_ref[0]
    n2q = jnp.sum(cq * cq, axis=1, keepdims=True)
    n2m = jnp.sum(cT * cT, axis=0, keepdims=True)
    d2 = n2q + n2m - 2.0 * _bdot(cq, cT)
    g = jnp.full((R, 1), -jnp.inf, jnp.float32)
    for _ in range(k):
        g = jnp.min(jnp.where(d2 > g, d2, jnp.inf), axis=1, keepdims=True)
    mask = d2 <= g
    ht = ht_ref[0]
    hf = hf_ref[0]
    hT = hT_ref[0]
    wqkv = wqkv_ref[...]
    q = _bdot(ht, wqkv[:, 0:d])
    outs = []
    for hh in range(heads):
        wk_h = wqkv[:, d + hh * dh:d + (hh + 1) * dh]
        wv_h = wqkv[:, 2 * d + hh * dh:2 * d + (hh + 1) * dh]
        kTh = _bdot(wk_h.T, hT)
        vh = _bdot(hf, wv_h)
        lg = _bdot(q[:, hh * dh:(hh + 1) * dh], kTh) * scale
        ml = jnp.where(mask, lg, _NEG)
        mx = jnp.max(ml, axis=1, keepdims=True)
        e = jnp.where(mask, jnp.exp(ml - mx), 0.0)
        s = jnp.sum(e, axis=1, keepdims=True)
        outs.append(_bdot(e / s, vh))
    out = jnp.concatenate(outs, axis=1)
    o_ref[0] = ht + _bdot(out, wo_ref[...])


def _attention(h, c, wqkv, wo, k, R):
    b, N, d = h.shape
    hT = jnp.swapaxes(h, 1, 2)
    cT = jnp.swapaxes(c, 1, 2)
    body = functools.partial(_attn_body, k, _HEADS, d, R, N)
    return pl.pallas_call(
        body,
        grid=(b, N // R),
        in_specs=[
            pl.BlockSpec((1, R, d), lambda bi, ri: (bi, ri, 0)),
            pl.BlockSpec((1, N, d), lambda bi, ri: (bi, 0, 0)),
            pl.BlockSpec((1, d, N), lambda bi, ri: (bi, 0, 0)),
            pl.BlockSpec((1, R, 3), lambda bi, ri: (bi, ri, 0)),
            pl.BlockSpec((1, 3, N), lambda bi, ri: (bi, 0, 0)),
            pl.BlockSpec((d, 3 * d), lambda bi, ri: (0, 0)),
            pl.BlockSpec((d, d), lambda bi, ri: (0, 0)),
        ],
        out_specs=pl.BlockSpec((1, R, d), lambda bi, ri: (bi, ri, 0)),
        out_shape=jax.ShapeDtypeStruct((b, N, d), jnp.float32),
        compiler_params=_par(2),
    )(h, h, hT, c, cT, wqkv, wo)


def _fps_body(npoint, N, cp_ref, c_ref, nc_ref):
    sub = N // 128
    lane = jax.lax.broadcasted_iota(jnp.int32, (sub, 128), 1)
    slid = jax.lax.broadcasted_iota(jnp.int32, (sub, 128), 0)
    flat = slid * 128 + lane
    big = jnp.int32(2 ** 31 - 1)
    nc_ref[0, 0:1, :] = c_ref[0, 0:1, :]
    cx = cp_ref[0, 0]
    cy = cp_ref[0, 1]
    cz = cp_ref[0, 2]

    def body(i, state):
        dist, lx, ly, lz = state
        dx = cx - lx
        dy = cy - ly
        dz = cz - lz
        d2 = dx * dx + dy * dy + dz * dz
        dist = jnp.minimum(dist, d2)
        m = jnp.max(jnp.max(dist, axis=0, keepdims=True),
                    axis=1, keepdims=True)
        cand = jnp.where(dist == m, flat, big)
        nxt = jnp.min(jnp.min(cand, axis=0, keepdims=True),
                      axis=1, keepdims=True)
        hot = flat == nxt
        lx = jnp.sum(jnp.sum(jnp.where(hot, cx, 0.0), axis=0, keepdims=True),
                     axis=1, keepdims=True)
        ly = jnp.sum(jnp.sum(jnp.where(hot, cy, 0.0), axis=0, keepdims=True),
                     axis=1, keepdims=True)
        lz = jnp.sum(jnp.sum(jnp.where(hot, cz, 0.0), axis=0, keepdims=True),
                     axis=1, keepdims=True)
        nc_ref[0, pl.ds(i, 1), :] = jnp.concatenate([lx, ly, lz], axis=1)
        return (dist, lx, ly, lz)

    dist0 = jnp.full((sub, 128), 1e10, jnp.float32)
    jax.lax.fori_loop(1, npoint, body,
                      (dist0, cx[0:1, 0:1], cy[0:1, 0:1], cz[0:1, 0:1]))


def _fps(c, npoint):
    b, N, _ = c.shape
    cp = jnp.swapaxes(c, 1, 2).reshape(b, 3, N // 128, 128)
    body = functools.partial(_fps_body, npoint, N)
    return pl.pallas_call(
        body,
        grid=(b,),
        in_specs=[
            pl.BlockSpec((1, 3, N // 128, 128), lambda bi: (bi, 0, 0, 0)),
            pl.BlockSpec((1, N, 3), lambda bi: (bi, 0, 0)),
        ],
        out_specs=pl.BlockSpec((1, npoint, 3), lambda bi: (bi, 0, 0)),
        out_shape=jax.ShapeDtypeStruct((b, npoint, 3), jnp.float32),
        compiler_params=_par(1),
    )(cp, c)


def _feat_body(ns, dout, Rq, N,
               nc_ref, c_ref, cT_ref, hf_ref, wx_ref, wc_ref, b_ref, o_ref):
    nc = nc_ref[0]
    cT = cT_ref[0]
    n2q = jnp.sum(nc * nc, axis=1, keepdims=True)
    n2m = jnp.sum(cT * cT, axis=0, keepdims=True)
    d2 = n2q + n2m - 2.0 * _bdot(nc, cT)
    pv = _bdot(hf_ref[0], wx_ref[...]) + _bdot(c_ref[0], wc_ref[...])
    g = jnp.full((Rq, 1), -jnp.inf, jnp.float32)
    acc = jnp.full((Rq, dout), _NEG, jnp.float32)
    for _ in range(ns):
        g = jnp.min(jnp.where(d2 > g, d2, jnp.inf), axis=1, keepdims=True)
        e = (d2 == g).astype(jnp.float32)
        acc = jnp.maximum(acc, jnp.dot(e, pv,
                                       preferred_element_type=jnp.float32,
                                       precision=_HI))
    ncp = _bdot(nc, wc_ref[...])
    o_ref[0] = jnp.maximum(acc - ncp + b_ref[...], 0.0)


def _fps_embed(c, h, npoint, w, bb, Rq):
    b, N, d = h.shape
    dout = w.shape[1]
    wx = w[0:d]
    wc = w[d:d + 3]
    nc = _fps(c, npoint)
    cT = jnp.swapaxes(c, 1, 2)
    body = functools.partial(_feat_body, _NS, dout, Rq, N)
    feat = pl.pallas_call(
        body,
        grid=(b, npoint // Rq),
        in_specs=[
            pl.BlockSpec((1, Rq, 3), lambda bi, ri: (bi, ri, 0)),
            pl.BlockSpec((1, N, 3), lambda bi, ri: (bi, 0, 0)),
            pl.BlockSpec((1, 3, N), lambda bi, ri: (bi, 0, 0)),
            pl.BlockSpec((1, N, d), lambda bi, ri: (bi, 0, 0)),
            pl.BlockSpec((d, dout), lambda bi, ri: (0, 0)),
            pl.BlockSpec((3, dout), lambda bi, ri: (0, 0)),
            pl.BlockSpec((1, dout), lambda bi, ri: (0, 0)),
        ],
        out_specs=pl.BlockSpec((1, Rq, dout), lambda bi, ri: (bi, ri, 0)),
        out_shape=jax.ShapeDtypeStruct((b, npoint, dout), jnp.float32),
        compiler_params=_par(2),
    )(nc, c, cT, h, wx, wc, bb.reshape(1, dout))
    return feat, nc


_ATTN_R = [512, 512, 256, 64]
_FEAT_R = [256, 256, 64]


def kernel(coords, x, W_stfe, b_stfe, blk0_Wqkv, blk0_Wo, blk1_Wqkv, blk1_Wo,
           blk2_Wqkv, blk2_Wo, blk3_Wqkv, blk3_Wo, fps0_W, fps0_b, fps1_W,
           fps1_b, fps2_W, fps2_b):
    wqkv = [blk0_Wqkv, blk1_Wqkv, blk2_Wqkv, blk3_Wqkv]
    wo = [blk0_Wo, blk1_Wo, blk2_Wo, blk3_Wo]
    fw = [fps0_W, fps1_W, fps2_W]
    fb = [fps0_b, fps1_b, fps2_b]
    b, N = x.shape[0], x.shape[1]
    h = _mm_bias(x.reshape(b * N, -1), W_stfe, b_stfe).reshape(b, N, _EMBED[0])
    c = coords[:, :, 0:3]
    for i in range(4):
        h = _attention(h, c, wqkv[i], wo[i], _KNN[i], _ATTN_R[i])
        if i != 3:
            npoint = _N0 // 4 ** (i + 1)
            h, c = _fps_embed(c, h, npoint, fw[i], fb[i], _FEAT_R[i])
            N = npoint
    return h


# speedup vs baseline: 14.6489x; 1.2120x over previous
import functools
import math

import jax
import jax.numpy as jnp
from jax.experimental import pallas as pl
from jax.experimental.pallas import tpu as pltpu

_EMBED = [64, 128, 256, 512]
_N0 = 4096
_HEADS = 2
_KNN = [20, 20, 20, 8]
_NS = 16
_HI = jax.lax.Precision.HIGHEST
_NEG = -1e30
_BF = jnp.bfloat16


def _bdot(a, b):
    return jnp.dot(a.astype(_BF), b.astype(_BF),
                   preferred_element_type=jnp.float32)


def _mm_kern(x_ref, w_ref, o_ref):
    o_ref[...] = _bdot(x_ref[...], w_ref[...])


def _mm_bias_kern(x_ref, w_ref, b_ref, o_ref):
    o_ref[...] = _bdot(x_ref[...], w_ref[...]) + b_ref[...]


def _par(n):
    return pltpu.CompilerParams(dimension_semantics=("parallel",) * n)


def _mm(x, w):
    m = x.shape[0]
    return pl.pallas_call(
        _mm_kern,
        grid=(2,),
        in_specs=[
            pl.BlockSpec((m // 2, x.shape[1]), lambda i: (i, 0)),
            pl.BlockSpec((x.shape[1], w.shape[1]), lambda i: (0, 0)),
        ],
        out_specs=pl.BlockSpec((m // 2, w.shape[1]), lambda i: (i, 0)),
        out_shape=jax.ShapeDtypeStruct((m, w.shape[1]), jnp.float32),
        compiler_params=_par(1),
    )(x, w)


def _mm_bias(x, w, b):
    m = x.shape[0]
    return pl.pallas_call(
        _mm_bias_kern,
        grid=(2,),
        in_specs=[
            pl.BlockSpec((m // 2, x.shape[1]), lambda i: (i, 0)),
            pl.BlockSpec((x.shape[1], w.shape[1]), lambda i: (0, 0)),
            pl.BlockSpec((1, w.shape[1]), lambda i: (0, 0)),
        ],
        out_specs=pl.BlockSpec((m // 2, w.shape[1]), lambda i: (i, 0)),
        out_shape=jax.ShapeDtypeStruct((m, w.shape[1]), jnp.float32),
        compiler_params=_par(1),
    )(x, w, b.reshape(1, -1))


def _pv_kern(h_ref, c_ref, wx_ref, wc_ref, o_ref):
    o_ref[...] = _bdot(h_ref[...], wx_ref[...]) + _bdot(c_ref[...], wc_ref[...])


def _pv(h2, c2, wx, wc):
    m = h2.shape[0]
    return pl.pallas_call(
        _pv_kern,
        grid=(2,),
        in_specs=[
            pl.BlockSpec((m // 2, h2.shape[1]), lambda i: (i, 0)),
            pl.BlockSpec((m // 2, 3), lambda i: (i, 0)),
            pl.BlockSpec((h2.shape[1], wx.shape[1]), lambda i: (0, 0)),
            pl.BlockSpec((3, wx.shape[1]), lambda i: (0, 0)),
        ],
        out_specs=pl.BlockSpec((m // 2, wx.shape[1]), lambda i: (i, 0)),
        out_shape=jax.ShapeDtypeStruct((m, wx.shape[1]), jnp.float32),
        compiler_params=_par(1),
    )(h2, c2, wx, wc)


def _attn_body(k, heads, d, R, N,
               ht_ref, hf_ref, hT_ref, cq_ref, cT_ref, wqkv_ref, wo_ref,
               o_ref):
    dh = d // heads
    scale = 1.0 / math.sqrt(float(dh))
    cq = cq_ref[0]
    cT = cT_ref[0]
    n2q = jnp.sum(cq * cq, axis=1, keepdims=True)
    n2m = jnp.sum(cT * cT, axis=0, keepdims=True)
    d2 = n2q + n2m - 2.0 * _bdot(cq, cT)
    g = jnp.full((R, 1), -jnp.inf, jnp.float32)
    for _ in range(k):
        g = jnp.min(jnp.where(d2 > g, d2, jnp.inf), axis=1, keepdims=True)
    mask = d2 <= g
    ht = ht_ref[0]
    hf = hf_ref[0]
    hT = hT_ref[0]
    wqkv = wqkv_ref[...]
    q = _bdot(ht, wqkv[:, 0:d])
    outs = []
    for hh in range(heads):
        wk_h = wqkv[:, d + hh * dh:d + (hh + 1) * dh]
        wv_h = wqkv[:, 2 * d + hh * dh:2 * d + (hh + 1) * dh]
        kTh = _bdot(wk_h.T, hT)
        vh = _bdot(hf, wv_h)
        lg = _bdot(q[:, hh * dh:(hh + 1) * dh], kTh) * scale
        ml = jnp.where(mask, lg, _NEG)
        mx = jnp.max(ml, axis=1, keepdims=True)
        e = jnp.where(mask, jnp.exp(ml - mx), 0.0)
        s = jnp.sum(e, axis=1, keepdims=True)
        outs.append(_bdot(e / s, vh))
    out = jnp.concatenate(outs, axis=1)
    o_ref[0] = ht + _bdot(out, wo_ref[...])


def _attention(h, c, wqkv, wo, k, R):
    b, N, d = h.shape
    hT = jnp.swapaxes(h, 1, 2)
    cT = jnp.swapaxes(c, 1, 2)
    body = functools.partial(_attn_body, k, _HEADS, d, R, N)
    return pl.pallas_call(
        body,
        grid=(b, N // R),
        in_specs=[
            pl.BlockSpec((1, R, d), lambda bi, ri: (bi, ri, 0)),
            pl.BlockSpec((1, N, d), lambda bi, ri: (bi, 0, 0)),
            pl.BlockSpec((1, d, N), lambda bi, ri: (bi, 0, 0)),
            pl.BlockSpec((1, R, 3), lambda bi, ri: (bi, ri, 0)),
            pl.BlockSpec((1, 3, N), lambda bi, ri: (bi, 0, 0)),
            pl.BlockSpec((d, 3 * d), lambda bi, ri: (0, 0)),
            pl.BlockSpec((d, d), lambda bi, ri: (0, 0)),
        ],
        out_specs=pl.BlockSpec((1, R, d), lambda bi, ri: (bi, ri, 0)),
        out_shape=jax.ShapeDtypeStruct((b, N, d), jnp.float32),
        compiler_params=_par(2),
    )(h, h, hT, c, cT, wqkv, wo)


def _fps_body(npoint, N, cp_ref, c_ref, nc_ref):
    sub = N // 128
    lane = jax.lax.broadcasted_iota(jnp.int32, (sub, 128), 1)
    slid = jax.lax.broadcasted_iota(jnp.int32, (sub, 128), 0)
    flat = slid * 128 + lane
    big = jnp.int32(2 ** 31 - 1)
    nc_ref[0, 0:1, :] = c_ref[0, 0:1, :]
    cx = cp_ref[0, 0]
    cy = cp_ref[0, 1]
    cz = cp_ref[0, 2]

    def body(i, state):
        dist, lx, ly, lz = state
        dx = cx - lx
        dy = cy - ly
        dz = cz - lz
        d2 = dx * dx + dy * dy + dz * dz
        dist = jnp.minimum(dist, d2)
        m = jnp.max(jnp.max(dist, axis=0, keepdims=True),
                    axis=1, keepdims=True)
        cand = jnp.where(dist == m, flat, big)
        nxt = jnp.min(jnp.min(cand, axis=0, keepdims=True),
                      axis=1, keepdims=True)
        hot = flat == nxt
        lx = jnp.sum(jnp.sum(jnp.where(hot, cx, 0.0), axis=0, keepdims=True),
                     axis=1, keepdims=True)
        ly = jnp.sum(jnp.sum(jnp.where(hot, cy, 0.0), axis=0, keepdims=True),
                     axis=1, keepdims=True)
        lz = jnp.sum(jnp.sum(jnp.where(hot, cz, 0.0), axis=0, keepdims=True),
                     axis=1, keepdims=True)
        nc_ref[0, pl.ds(i, 1), :] = jnp.concatenate([lx, ly, lz], axis=1)
        return (dist, lx, ly, lz)

    dist0 = jnp.full((sub, 128), 1e10, jnp.float32)
    jax.lax.fori_loop(1, npoint, body,
                      (dist0, cx[0:1, 0:1], cy[0:1, 0:1], cz[0:1, 0:1]))


def _fps(c, npoint):
    b, N, _ = c.shape
    cp = jnp.swapaxes(c, 1, 2).reshape(b, 3, N // 128, 128)
    body = functools.partial(_fps_body, npoint, N)
    return pl.pallas_call(
        body,
        grid=(b,),
        in_specs=[
            pl.BlockSpec((1, 3, N // 128, 128), lambda bi: (bi, 0, 0, 0)),
            pl.BlockSpec((1, N, 3), lambda bi: (bi, 0, 0)),
        ],
        out_specs=pl.BlockSpec((1, npoint, 3), lambda bi: (bi, 0, 0)),
        out_shape=jax.ShapeDtypeStruct((b, npoint, 3), jnp.float32),
        compiler_params=_par(1),
    )(cp, c)


def _feat_body(ns, dout, Rq, N,
               nc_ref, c_ref, cT_ref, hf_ref, wx_ref, wc_ref, b_ref, o_ref):
    nc = nc_ref[0]
    cT = cT_ref[0]
    n2q = jnp.sum(nc * nc, axis=1, keepdims=True)
    n2m = jnp.sum(cT * cT, axis=0, keepdims=True)
    d2 = n2q + n2m - 2.0 * _bdot(nc, cT)
    pv = _bdot(hf_ref[0], wx_ref[...]) + _bdot(c_ref[0], wc_ref[...])
    pv_hi = pv.astype(_BF)
    pv_lo = (pv - pv_hi.astype(jnp.float32)).astype(_BF)
    g = jnp.full((Rq, 1), -jnp.inf, jnp.float32)
    acc = jnp.full((Rq, dout), _NEG, jnp.float32)
    for _ in range(ns):
        g = jnp.min(jnp.where(d2 > g, d2, jnp.inf), axis=1, keepdims=True)
        e = (d2 == g).astype(_BF)
        acc = jnp.maximum(
            acc,
            jnp.dot(e, pv_hi, preferred_element_type=jnp.float32)
            + jnp.dot(e, pv_lo, preferred_element_type=jnp.float32))
    ncp = _bdot(nc, wc_ref[...])
    o_ref[0] = jnp.maximum(acc - ncp + b_ref[...], 0.0)


def _fps_embed(c, h, npoint, w, bb, Rq):
    b, N, d = h.shape
    dout = w.shape[1]
    wx = w[0:d]
    wc = w[d:d + 3]
    nc = _fps(c, npoint)
    cT = jnp.swapaxes(c, 1, 2)
    body = functools.partial(_feat_body, _NS, dout, Rq, N)
    feat = pl.pallas_call(
        body,
        grid=(b, npoint // Rq),
        in_specs=[
            pl.BlockSpec((1, Rq, 3), lambda bi, ri: (bi, ri, 0)),
            pl.BlockSpec((1, N, 3), lambda bi, ri: (bi, 0, 0)),
            pl.BlockSpec((1, 3, N), lambda bi, ri: (bi, 0, 0)),
            pl.BlockSpec((1, N, d), lambda bi, ri: (bi, 0, 0)),
            pl.BlockSpec((d, dout), lambda bi, ri: (0, 0)),
            pl.BlockSpec((3, dout), lambda bi, ri: (0, 0)),
            pl.BlockSpec((1, dout), lambda bi, ri: (0, 0)),
        ],
        out_specs=pl.BlockSpec((1, Rq, dout), lambda bi, ri: (bi, ri, 0)),
        out_shape=jax.ShapeDtypeStruct((b, npoint, dout), jnp.float32),
        compiler_params=_par(2),
    )(nc, c, cT, h, wx, wc, bb.reshape(1, dout))
    return feat, nc


_ATTN_R = [512, 512, 256, 64]
_FEAT_R = [256, 256, 64]


def kernel(coords, x, W_stfe, b_stfe, blk0_Wqkv, blk0_Wo, blk1_Wqkv, blk1_Wo,
           blk2_Wqkv, blk2_Wo, blk3_Wqkv, blk3_Wo, fps0_W, fps0_b, fps1_W,
           fps1_b, fps2_W, fps2_b):
    wqkv = [blk0_Wqkv, blk1_Wqkv, blk2_Wqkv, blk3_Wqkv]
    wo = [blk0_Wo, blk1_Wo, blk2_Wo, blk3_Wo]
    fw = [fps0_W, fps1_W, fps2_W]
    fb = [fps0_b, fps1_b, fps2_b]
    b, N = x.shape[0], x.shape[1]
    h = _mm_bias(x.reshape(b * N, -1), W_stfe, b_stfe).reshape(b, N, _EMBED[0])
    c = coords[:, :, 0:3]
    for i in range(4):
        h = _attention(h, c, wqkv[i], wo[i], _KNN[i], _ATTN_R[i])
        if i != 3:
            npoint = _N0 // 4 ** (i + 1)
            h, c = _fps_embed(c, h, npoint, fw[i], fb[i], _FEAT_R[i])
            N = npoint
    return h
